# baseline (speedup 1.0000x reference)
import functools

import jax
import jax.numpy as jnp
from jax import lax
from jax.experimental import pallas as pl
from jax.experimental.pallas import tpu as pltpu
from jax.experimental.pallas import tpu_sc as plsc

_NUM_CORES = 2
_NUM_SUBCORES = 16
_NW = _NUM_CORES * _NUM_SUBCORES
_CHUNK = 32


def _sc_gather(table, idx):
    V, D = table.shape
    B = idx.shape[0]
    b_per_w = B // _NW
    n_chunks = b_per_w // _CHUNK
    mesh = plsc.VectorSubcoreMesh(core_axis_name="c", subcore_axis_name="s")

    @functools.partial(
        pl.kernel,
        mesh=mesh,
        out_type=jax.ShapeDtypeStruct((B, D), jnp.float32),
        scratch_types=[
            pltpu.VMEM((_CHUNK,), jnp.int32),
            pltpu.VMEM((_CHUNK, D), jnp.float32),
            pltpu.SemaphoreType.DMA,
        ],
    )
    def k(table_hbm, idx_hbm, out_hbm, idx_v, rows_v, sem):
        wid = lax.axis_index("s") * _NUM_CORES + lax.axis_index("c")
        base = wid * b_per_w
        for ci in range(n_chunks):
            off = pl.multiple_of(base + ci * _CHUNK, _CHUNK)
            pltpu.sync_copy(idx_hbm.at[pl.ds(off, _CHUNK)], idx_v)
            pltpu.async_copy(table_hbm.at[idx_v], rows_v, sem).wait()
            pltpu.sync_copy(rows_v, out_hbm.at[pl.ds(off, _CHUNK)])

    return k(table, idx)


_BB = 8


def _tc_body(vl_ref, tl_ref, cls_ref, pref_ref, s1_ref, s2_ref, s3_ref,
             other_ref, out_ref):
    i = pl.program_id(0)
    bb = out_ref.shape[0]
    out_ref[:, 0:5, :] = jnp.broadcast_to(pref_ref[0], (bb, 5, 512))
    out_ref[:, 5:9, :] = cls_ref[...]
    out_ref[:, 9:11, :] = jnp.broadcast_to(s1_ref[0], (bb, 2, 512))
    out_ref[:, 12:14, :] = jnp.broadcast_to(s2_ref[0], (bb, 2, 512))
    out_ref[:, 15:77, :] = jnp.broadcast_to(s3_ref[0], (bb, 62, 512))
    o0 = other_ref[0, 0, :]
    o1 = other_ref[0, 1, :]
    o2 = other_ref[0, 2, :]
    o3 = other_ref[0, 3, :]
    for k in range(bb):
        vl = vl_ref[i * bb + k]
        tl = tl_ref[i * bb + k]
        out_ref[k, pl.ds(11, 1), :] = jnp.where(vl == 0, o0, o1)[None, :]
        out_ref[k, pl.ds(14, 1), :] = jnp.where(tl == 0, o2, o3)[None, :]


def _tc_assemble(vl, tl, cls, pref, s1, s2, s3, other):
    B = cls.shape[0]
    full = lambda shape: pl.BlockSpec(shape, lambda i: (0, 0, 0))
    return pl.pallas_call(
        _tc_body,
        grid=(B // _BB,),
        in_specs=[
            pl.BlockSpec(memory_space=pltpu.SMEM),
            pl.BlockSpec(memory_space=pltpu.SMEM),
            pl.BlockSpec((_BB, 4, 512), lambda i: (i, 0, 0)),
            full((1, 5, 512)),
            full((1, 2, 512)),
            full((1, 2, 512)),
            full((1, 62, 512)),
            full((1, 4, 512)),
        ],
        out_specs=pl.BlockSpec((_BB, 77, 512), lambda i: (i, 0, 0)),
        out_shape=jax.ShapeDtypeStruct((B, 77, 512), jnp.float32),
    )(vl, tl, cls, pref, s1, s2, s3, other)


def kernel(label, view_label, time_label, ctx_cls, token_prefix,
           token_suffix1, token_suffix2, token_suffix3, token_other):
    V, N, D = ctx_cls.shape
    B = label.shape[0]
    table = ctx_cls.reshape(V, N * D)
    cls_flat = _sc_gather(table, label.astype(jnp.int32))
    cls = cls_flat.reshape(B, N, D)
    return _tc_assemble(view_label.astype(jnp.int32),
                        time_label.astype(jnp.int32), cls, token_prefix,
                        token_suffix1, token_suffix2, token_suffix3,
                        token_other)


# speedup vs baseline: 1.7074x; 1.7074x over previous
import functools

import jax
import jax.numpy as jnp
from jax import lax
from jax.experimental import pallas as pl
from jax.experimental.pallas import tpu as pltpu
from jax.experimental.pallas import tpu_sc as plsc

_NUM_CORES = 2
_NUM_SUBCORES = 16
_NW = _NUM_CORES * _NUM_SUBCORES
_CHUNK = 32


def _sc_gather(table, idx):
    V, N, D = table.shape
    B = idx.shape[0]
    b_per_w = B // _NW
    n_chunks = b_per_w // _CHUNK
    mesh = plsc.VectorSubcoreMesh(core_axis_name="c", subcore_axis_name="s")

    @functools.partial(
        pl.kernel,
        mesh=mesh,
        out_type=jax.ShapeDtypeStruct((B, N, D), jnp.float32),
        scratch_types=[
            pltpu.VMEM((_CHUNK,), jnp.int32),
            pltpu.VMEM((_CHUNK, N, D), jnp.float32),
            pltpu.SemaphoreType.DMA,
        ],
        compiler_params=pltpu.CompilerParams(use_tc_tiling_on_sc=True),
    )
    def k(table_hbm, idx_hbm, out_hbm, idx_v, rows_v, sem):
        wid = lax.axis_index("s") * _NUM_CORES + lax.axis_index("c")
        base = wid * b_per_w
        for ci in range(n_chunks):
            off = pl.multiple_of(base + ci * _CHUNK, _CHUNK)
            pltpu.sync_copy(idx_hbm.at[pl.ds(off, _CHUNK)], idx_v)
            pltpu.async_copy(table_hbm.at[idx_v], rows_v, sem).wait()
            pltpu.sync_copy(rows_v, out_hbm.at[pl.ds(off, _CHUNK)])

    return k(table, idx)


_BB = 8


def _tc_body(vl_ref, tl_ref, cls_ref, pref_ref, s1_ref, s2_ref, s3_ref,
             other_ref, out_ref):
    i = pl.program_id(0)
    bb = out_ref.shape[0]
    out_ref[:, 0:5, :] = jnp.broadcast_to(pref_ref[0], (bb, 5, 512))
    out_ref[:, 5:9, :] = cls_ref[...]
    out_ref[:, 9:11, :] = jnp.broadcast_to(s1_ref[0], (bb, 2, 512))
    out_ref[:, 12:14, :] = jnp.broadcast_to(s2_ref[0], (bb, 2, 512))
    out_ref[:, 15:77, :] = jnp.broadcast_to(s3_ref[0], (bb, 62, 512))
    o0 = other_ref[0, 0, :]
    o1 = other_ref[0, 1, :]
    o2 = other_ref[0, 2, :]
    o3 = other_ref[0, 3, :]
    for k in range(bb):
        vl = vl_ref[i * bb + k]
        tl = tl_ref[i * bb + k]
        out_ref[k, pl.ds(11, 1), :] = jnp.where(vl == 0, o0, o1)[None, :]
        out_ref[k, pl.ds(14, 1), :] = jnp.where(tl == 0, o2, o3)[None, :]


def _tc_assemble(vl, tl, cls, pref, s1, s2, s3, other):
    B = cls.shape[0]
    full = lambda shape: pl.BlockSpec(shape, lambda i: (0, 0, 0))
    return pl.pallas_call(
        _tc_body,
        grid=(B // _BB,),
        in_specs=[
            pl.BlockSpec(memory_space=pltpu.SMEM),
            pl.BlockSpec(memory_space=pltpu.SMEM),
            pl.BlockSpec((_BB, 4, 512), lambda i: (i, 0, 0)),
            full((1, 5, 512)),
            full((1, 2, 512)),
            full((1, 2, 512)),
            full((1, 62, 512)),
            full((1, 4, 512)),
        ],
        out_specs=pl.BlockSpec((_BB, 77, 512), lambda i: (i, 0, 0)),
        out_shape=jax.ShapeDtypeStruct((B, 77, 512), jnp.float32),
    )(vl, tl, cls, pref, s1, s2, s3, other)


def kernel(label, view_label, time_label, ctx_cls, token_prefix,
           token_suffix1, token_suffix2, token_suffix3, token_other):
    cls = _sc_gather(ctx_cls, label.astype(jnp.int32))
    return _tc_assemble(view_label.astype(jnp.int32),
                        time_label.astype(jnp.int32), cls, token_prefix,
                        token_suffix1, token_suffix2, token_suffix3,
                        token_other)


# speedup vs baseline: 5.7544x; 3.3702x over previous
import functools

import jax
import jax.numpy as jnp
from jax import lax
from jax.experimental import pallas as pl
from jax.experimental.pallas import tpu as pltpu
from jax.experimental.pallas import tpu_sc as plsc

_NUM_CORES = 2
_NUM_SUBCORES = 16
_NW = _NUM_CORES * _NUM_SUBCORES
_CHUNK = 32


def _sc_gather(table, idx):
    V, N, D = table.shape
    B = idx.shape[0]
    b_per_w = B // _NW
    n_chunks = b_per_w // _CHUNK
    mesh = plsc.VectorSubcoreMesh(core_axis_name="c", subcore_axis_name="s")

    @functools.partial(
        pl.kernel,
        mesh=mesh,
        out_type=jax.ShapeDtypeStruct((N, B, D), jnp.float32),
        scratch_types=[
            pltpu.VMEM((_CHUNK,), jnp.int32),
            pltpu.VMEM((_CHUNK, N, D), jnp.float32),
            pltpu.SemaphoreType.DMA,
        ],
        compiler_params=pltpu.CompilerParams(use_tc_tiling_on_sc=True),
    )
    def k(table_hbm, idx_hbm, out_hbm, idx_v, rows_v, sem):
        wid = lax.axis_index("s") * _NUM_CORES + lax.axis_index("c")
        base = wid * b_per_w
        for ci in range(n_chunks):
            off = pl.multiple_of(base + ci * _CHUNK, _CHUNK)
            pltpu.sync_copy(idx_hbm.at[pl.ds(off, _CHUNK)], idx_v)
            pltpu.async_copy(table_hbm.at[idx_v], rows_v, sem).wait()
            for j in range(N):
                pltpu.sync_copy(rows_v.at[:, j],
                                out_hbm.at[j, pl.ds(off, _CHUNK)])

    return k(table, idx)


_BB = 64


def _tc_body(vl_ref, tl_ref, cls_ref, pref_ref, s1_ref, s2_ref, s3_ref,
             other_ref, out_ref):
    bb = out_ref.shape[1]
    row = lambda ref, t: ref[pl.ds(0, 1), pl.ds(t, 1), :]
    bcast = lambda ref, t: jnp.broadcast_to(row(ref, t), (1, bb, 512))
    for t in range(5):
        out_ref[pl.ds(t, 1)] = bcast(pref_ref, t)
    for j in range(4):
        out_ref[pl.ds(5 + j, 1)] = cls_ref[pl.ds(j, 1)]
    out_ref[pl.ds(9, 1)] = bcast(s1_ref, 0)
    out_ref[pl.ds(10, 1)] = bcast(s1_ref, 1)
    vmask = jnp.broadcast_to(vl_ref[...] == 0, (1, bb, 512))
    out_ref[pl.ds(11, 1)] = jnp.where(vmask, bcast(other_ref, 0),
                                      bcast(other_ref, 1))
    out_ref[pl.ds(12, 1)] = bcast(s2_ref, 0)
    out_ref[pl.ds(13, 1)] = bcast(s2_ref, 1)
    tmask = jnp.broadcast_to(tl_ref[...] == 0, (1, bb, 512))
    out_ref[pl.ds(14, 1)] = jnp.where(tmask, bcast(other_ref, 2),
                                      bcast(other_ref, 3))
    for t in range(62):
        out_ref[pl.ds(15 + t, 1)] = bcast(s3_ref, t)


def _tc_assemble(vl, tl, cls_t, pref, s1, s2, s3, other):
    B = cls_t.shape[1]
    nb = B // _BB
    full = lambda shape: pl.BlockSpec(shape, lambda i: (0, 0, 0))
    out_t = pl.pallas_call(
        _tc_body,
        grid=(nb,),
        in_specs=[
            pl.BlockSpec((1, _BB, 1), lambda i: (i, 0, 0)),
            pl.BlockSpec((1, _BB, 1), lambda i: (i, 0, 0)),
            pl.BlockSpec((4, _BB, 512), lambda i: (0, i, 0)),
            full((1, 5, 512)),
            full((1, 2, 512)),
            full((1, 2, 512)),
            full((1, 62, 512)),
            full((1, 4, 512)),
        ],
        out_specs=pl.BlockSpec((77, _BB, 512), lambda i: (0, i, 0)),
        out_shape=jax.ShapeDtypeStruct((77, B, 512), jnp.float32),
    )(vl.reshape(nb, _BB, 1), tl.reshape(nb, _BB, 1), cls_t, pref, s1, s2,
      s3, other)
    return jnp.transpose(out_t, (1, 0, 2))


def kernel(label, view_label, time_label, ctx_cls, token_prefix,
           token_suffix1, token_suffix2, token_suffix3, token_other):
    cls = _sc_gather(ctx_cls, label.astype(jnp.int32))
    return _tc_assemble(view_label.astype(jnp.int32),
                        time_label.astype(jnp.int32), cls, token_prefix,
                        token_suffix1, token_suffix2, token_suffix3,
                        token_other)


# speedup vs baseline: 5.8306x; 1.0132x over previous
import functools

import jax
import jax.numpy as jnp
from jax import lax
from jax.experimental import pallas as pl
from jax.experimental.pallas import tpu as pltpu
from jax.experimental.pallas import tpu_sc as plsc

_NUM_CORES = 2
_NUM_SUBCORES = 16
_NW = _NUM_CORES * _NUM_SUBCORES
_CHUNK = 32


def _sc_gather(table, idx):
    V, N, D = table.shape
    B = idx.shape[0]
    b_per_w = B // _NW
    n_chunks = b_per_w // _CHUNK
    mesh = plsc.VectorSubcoreMesh(core_axis_name="c", subcore_axis_name="s")

    @functools.partial(
        pl.kernel,
        mesh=mesh,
        out_type=jax.ShapeDtypeStruct((N, B, D), jnp.float32),
        scratch_types=[
            pltpu.VMEM((_CHUNK,), jnp.int32),
            pltpu.VMEM((_CHUNK, N, D), jnp.float32),
            pltpu.SemaphoreType.DMA,
        ],
        compiler_params=pltpu.CompilerParams(use_tc_tiling_on_sc=True),
    )
    def k(table_hbm, idx_hbm, out_hbm, idx_v, rows_v, sem):
        wid = lax.axis_index("s") * _NUM_CORES + lax.axis_index("c")
        base = wid * b_per_w
        for ci in range(n_chunks):
            off = pl.multiple_of(base + ci * _CHUNK, _CHUNK)
            pltpu.sync_copy(idx_hbm.at[pl.ds(off, _CHUNK)], idx_v)
            pltpu.async_copy(table_hbm.at[idx_v], rows_v, sem).wait()
            for j in range(N):
                pltpu.sync_copy(rows_v.at[:, j],
                                out_hbm.at[j, pl.ds(off, _CHUNK)])

    return k(table, idx)


_BB = 128


def _tc_body(vl_ref, tl_ref, cls_ref, pref_ref, s1_ref, s2_ref, s3_ref,
             other_ref, out_ref):
    bb = out_ref.shape[1]
    row = lambda ref, t: ref[pl.ds(0, 1), pl.ds(t, 1), :]
    bcast = lambda ref, t: jnp.broadcast_to(row(ref, t), (1, bb, 512))
    for t in range(5):
        out_ref[pl.ds(t, 1)] = bcast(pref_ref, t)
    for j in range(4):
        out_ref[pl.ds(5 + j, 1)] = cls_ref[pl.ds(j, 1)]
    out_ref[pl.ds(9, 1)] = bcast(s1_ref, 0)
    out_ref[pl.ds(10, 1)] = bcast(s1_ref, 1)
    vmask = jnp.broadcast_to(vl_ref[...] == 0, (1, bb, 512))
    out_ref[pl.ds(11, 1)] = jnp.where(vmask, bcast(other_ref, 0),
                                      bcast(other_ref, 1))
    out_ref[pl.ds(12, 1)] = bcast(s2_ref, 0)
    out_ref[pl.ds(13, 1)] = bcast(s2_ref, 1)
    tmask = jnp.broadcast_to(tl_ref[...] == 0, (1, bb, 512))
    out_ref[pl.ds(14, 1)] = jnp.where(tmask, bcast(other_ref, 2),
                                      bcast(other_ref, 3))
    for t in range(62):
        out_ref[pl.ds(15 + t, 1)] = bcast(s3_ref, t)


def _tc_assemble(vl, tl, cls_t, pref, s1, s2, s3, other):
    B = cls_t.shape[1]
    nb = B // _BB
    full = lambda shape: pl.BlockSpec(shape, lambda i: (0, 0, 0))
    out_t = pl.pallas_call(
        _tc_body,
        grid=(nb,),
        in_specs=[
            pl.BlockSpec((1, _BB, 1), lambda i: (i, 0, 0)),
            pl.BlockSpec((1, _BB, 1), lambda i: (i, 0, 0)),
            pl.BlockSpec((4, _BB, 512), lambda i: (0, i, 0)),
            full((1, 5, 512)),
            full((1, 2, 512)),
            full((1, 2, 512)),
            full((1, 62, 512)),
            full((1, 4, 512)),
        ],
        out_specs=pl.BlockSpec((77, _BB, 512), lambda i: (0, i, 0)),
        out_shape=jax.ShapeDtypeStruct((77, B, 512), jnp.float32),
    )(vl.reshape(nb, _BB, 1), tl.reshape(nb, _BB, 1), cls_t, pref, s1, s2,
      s3, other)
    return jnp.transpose(out_t, (1, 0, 2))


def kernel(label, view_label, time_label, ctx_cls, token_prefix,
           token_suffix1, token_suffix2, token_suffix3, token_other):
    cls = _sc_gather(ctx_cls, label.astype(jnp.int32))
    return _tc_assemble(view_label.astype(jnp.int32),
                        time_label.astype(jnp.int32), cls, token_prefix,
                        token_suffix1, token_suffix2, token_suffix3,
                        token_other)


# speedup vs baseline: 6.2907x; 1.0789x over previous
import functools

import jax
import jax.numpy as jnp
from jax import lax
from jax.experimental import pallas as pl
from jax.experimental.pallas import tpu as pltpu
from jax.experimental.pallas import tpu_sc as plsc

_NUM_CORES = 2
_NUM_SUBCORES = 16
_NW = _NUM_CORES * _NUM_SUBCORES
_CHUNK = 32

_T = 77
_CLS0 = 5


def _sc_gather(table, idx):
    V, N, D = table.shape
    B = idx.shape[0]
    b_per_w = B // _NW
    n_chunks = b_per_w // _CHUNK
    mesh = plsc.VectorSubcoreMesh(core_axis_name="c", subcore_axis_name="s")

    @functools.partial(
        pl.kernel,
        mesh=mesh,
        out_type=jax.ShapeDtypeStruct((_T, B, D), jnp.float32),
        scratch_types=[
            pltpu.VMEM((_CHUNK,), jnp.int32),
            pltpu.VMEM((_CHUNK, N, D), jnp.float32),
            pltpu.SemaphoreType.DMA,
        ],
        compiler_params=pltpu.CompilerParams(use_tc_tiling_on_sc=True),
    )
    def k(table_hbm, idx_hbm, out_hbm, idx_v, rows_v, sem):
        wid = lax.axis_index("s") * _NUM_CORES + lax.axis_index("c")
        base = wid * b_per_w
        for ci in range(n_chunks):
            off = pl.multiple_of(base + ci * _CHUNK, _CHUNK)
            pltpu.sync_copy(idx_hbm.at[pl.ds(off, _CHUNK)], idx_v)
            pltpu.async_copy(table_hbm.at[idx_v], rows_v, sem).wait()
            for j in range(N):
                pltpu.sync_copy(rows_v.at[:, j],
                                out_hbm.at[_CLS0 + j, pl.ds(off, _CHUNK)])

    return k(table, idx)


_BB = 4096


def _tc_body(buf_ref, static_ref, vl_ref, tl_ref, other_ref, out_ref):
    del buf_ref
    bb = out_ref.shape[1]
    t = pl.program_id(0)
    p = t + jnp.where(t >= _CLS0, 4, 0)
    base = jnp.broadcast_to(static_ref[pl.ds(0, 1), pl.ds(p, 1), :],
                            (1, bb, 512))
    alt_v = jnp.broadcast_to(other_ref[pl.ds(0, 1), pl.ds(1, 1), :],
                             (1, bb, 512))
    alt_t = jnp.broadcast_to(other_ref[pl.ds(0, 1), pl.ds(3, 1), :],
                             (1, bb, 512))
    vmask = jnp.broadcast_to((vl_ref[...] != 0) & (p == 11), (1, bb, 512))
    tmask = jnp.broadcast_to((tl_ref[...] != 0) & (p == 14), (1, bb, 512))
    out_ref[...] = jnp.where(tmask, alt_t, jnp.where(vmask, alt_v, base))


def _tc_assemble(buf, static_rows, vl, tl, other):
    B = buf.shape[1]

    def out_map(t):
        return (t + jnp.where(t >= _CLS0, 4, 0), 0, 0)

    out_t = pl.pallas_call(
        _tc_body,
        grid=(_T - 4,),
        in_specs=[
            pl.BlockSpec(memory_space=pl.ANY),
            pl.BlockSpec((1, _T, 512), lambda t: (0, 0, 0)),
            pl.BlockSpec((1, _BB, 1), lambda t: (0, 0, 0)),
            pl.BlockSpec((1, _BB, 1), lambda t: (0, 0, 0)),
            pl.BlockSpec((1, 4, 512), lambda t: (0, 0, 0)),
        ],
        out_specs=pl.BlockSpec((1, _BB, 512), out_map),
        out_shape=jax.ShapeDtypeStruct((_T, B, 512), jnp.float32),
        input_output_aliases={0: 0},
    )(buf, static_rows, vl.reshape(1, B, 1), tl.reshape(1, B, 1), other)
    return jnp.transpose(out_t, (1, 0, 2))


def kernel(label, view_label, time_label, ctx_cls, token_prefix,
           token_suffix1, token_suffix2, token_suffix3, token_other):
    buf = _sc_gather(ctx_cls, label.astype(jnp.int32))
    static_rows = jnp.concatenate([
        token_prefix,
        jnp.zeros((1, 4, 512), jnp.float32),
        token_suffix1,
        token_other[:, 0:1],
        token_suffix2,
        token_other[:, 2:3],
        token_suffix3,
    ], axis=1)
    return _tc_assemble(buf, static_rows, view_label.astype(jnp.int32),
                        time_label.astype(jnp.int32), token_other)
